# Initial kernel scaffold; baseline (speedup 1.0000x reference)
import jax
import jax.numpy as jnp
from jax.experimental import pallas as pl


def kernel(w_idx, c_idx, word_table, char_table, Wt0, bt0, Wg0, bg0, Wt1, bt1, Wg1, bg1):
    raise NotImplementedError("write your pallas kernel here")


# speedup vs baseline: 4.4395x; 4.4395x over previous
import functools

import jax
import jax.numpy as jnp
from jax import lax
from jax.experimental import pallas as pl
from jax.experimental.pallas import tpu as pltpu
from jax.experimental.pallas import tpu_sc as plsc

WORD_DIM = 64
CHAR_DIM = 16
H = 320
N_TOK = 1024 * 50
N_CHR = N_TOK * 16
NW = 32

IDX_MINOR = 100
W_ROWS = N_TOK // IDX_MINOR
C_ROWS = N_CHR // IDX_MINOR
W_ROWS_PER_WORKER = W_ROWS // NW
C_ROWS_PER_WORKER = C_ROWS // NW
W_CHUNK = 8
C_CHUNK = 32
W_SUPER = W_ROWS_PER_WORKER // W_CHUNK
C_SUPER = C_ROWS_PER_WORKER // C_CHUNK


def _sc_gather(widx2d, cidx2d, word_table, char_table):
    mesh = plsc.VectorSubcoreMesh(core_axis_name="c", subcore_axis_name="s")

    @functools.partial(
        pl.kernel,
        out_type=(
            jax.ShapeDtypeStruct((N_TOK, WORD_DIM), jnp.float32),
            jax.ShapeDtypeStruct((N_CHR, CHAR_DIM), jnp.float32),
        ),
        mesh=mesh,
        scratch_types=[
            pltpu.VMEM((W_CHUNK, IDX_MINOR), jnp.int32),
            pltpu.VMEM((W_CHUNK * IDX_MINOR, WORD_DIM), jnp.float32),
            pltpu.VMEM((C_CHUNK, IDX_MINOR), jnp.int32),
            pltpu.VMEM((C_CHUNK * IDX_MINOR, CHAR_DIM), jnp.float32),
            pltpu.SemaphoreType.DMA,
        ],
        compiler_params=pltpu.CompilerParams(use_tc_tiling_on_sc=False),
    )
    def gather_kernel(widx_hbm, cidx_hbm, wtab_hbm, ctab_hbm,
                      out_w, out_c, widx_v, wrows_v, cidx_v, crows_v, sem):
        wid = lax.axis_index("s") * 2 + lax.axis_index("c")

        def word_super(g, _):
            row0 = wid * W_ROWS_PER_WORKER + g * W_CHUNK
            pltpu.sync_copy(widx_hbm.at[pl.ds(row0, W_CHUNK)], widx_v)
            for j in range(W_CHUNK):
                pltpu.async_copy(
                    wtab_hbm.at[widx_v.at[j]],
                    wrows_v.at[pl.ds(j * IDX_MINOR, IDX_MINOR)],
                    sem,
                )
            pltpu.make_async_copy(
                out_w.at[pl.ds(0, W_CHUNK * IDX_MINOR)], wrows_v, sem
            ).wait()
            pltpu.sync_copy(
                wrows_v, out_w.at[pl.ds(row0 * IDX_MINOR, W_CHUNK * IDX_MINOR)]
            )
            return 0

        lax.fori_loop(0, W_SUPER, word_super, 0)

        def char_super(g, _):
            row0 = wid * C_ROWS_PER_WORKER + g * C_CHUNK
            pltpu.sync_copy(cidx_hbm.at[pl.ds(row0, C_CHUNK)], cidx_v)

            def char_octet(j, _):
                for k in range(8):
                    pltpu.async_copy(
                        ctab_hbm.at[cidx_v.at[j * 8 + k]],
                        crows_v.at[pl.ds((j * 8 + k) * IDX_MINOR, IDX_MINOR)],
                        sem,
                    )
                return 0

            lax.fori_loop(0, C_CHUNK // 8, char_octet, 0)
            pltpu.make_async_copy(
                out_c.at[pl.ds(0, C_CHUNK * IDX_MINOR)], crows_v, sem
            ).wait()
            pltpu.sync_copy(
                crows_v, out_c.at[pl.ds(row0 * IDX_MINOR, C_CHUNK * IDX_MINOR)]
            )
            return 0

        lax.fori_loop(0, C_SUPER, char_super, 0)

    return gather_kernel(widx2d, cidx2d, word_table, char_table)


BLK = 1024


def _highway_body(w_ref, c_ref, wt0, bt0, wg0, bg0, wt1, bt1, wg1, bg1, o_ref):
    x = jnp.concatenate([w_ref[...], c_ref[...]], axis=1)
    for wt, bt, wg, bg in ((wt0, bt0, wg0, bg0), (wt1, bt1, wg1, bg1)):
        g = jax.nn.sigmoid(
            jnp.dot(x, wg[...], preferred_element_type=jnp.float32) + bg[...]
        )
        t = jax.nn.relu(
            jnp.dot(x, wt[...], preferred_element_type=jnp.float32) + bt[...]
        )
        x = g * t + (1.0 - g) * x
    o_ref[...] = x


def _highway(word_rows, char_flat, wt0, bt0, wg0, bg0, wt1, bt1, wg1, bg1):
    grid = (N_TOK // BLK,)
    full = pl.BlockSpec((H, H), lambda i: (0, 0))
    vec = pl.BlockSpec((1, H), lambda i: (0, 0))
    return pl.pallas_call(
        _highway_body,
        grid=grid,
        in_specs=[
            pl.BlockSpec((BLK, WORD_DIM), lambda i: (i, 0)),
            pl.BlockSpec((BLK, H - WORD_DIM), lambda i: (i, 0)),
            full, vec, full, vec, full, vec, full, vec,
        ],
        out_specs=pl.BlockSpec((BLK, H), lambda i: (i, 0)),
        out_shape=jax.ShapeDtypeStruct((N_TOK, H), jnp.float32),
    )(word_rows, char_flat, wt0, bt0, wg0, bg0, wt1, bt1, wg1, bg1)


def kernel(w_idx, c_idx, word_table, char_table,
           Wt0, bt0, Wg0, bg0, Wt1, bt1, Wg1, bg1):
    B, L = w_idx.shape
    widx2d = w_idx.reshape(W_ROWS, IDX_MINOR).astype(jnp.int32)
    cidx2d = c_idx.reshape(C_ROWS, IDX_MINOR).astype(jnp.int32)

    word_rows, char_rows = _sc_gather(widx2d, cidx2d, word_table, char_table)
    char_flat = char_rows.reshape(N_TOK, H - WORD_DIM)

    out = _highway(
        word_rows, char_flat,
        Wt0.T, bt0.reshape(1, H), Wg0.T, bg0.reshape(1, H),
        Wt1.T, bt1.reshape(1, H), Wg1.T, bg1.reshape(1, H),
    )
    return out.reshape(B, L, H)
